# baseline (speedup 1.0000x reference)
import functools

import jax
import jax.numpy as jnp
from jax import lax
from jax.experimental import pallas as pl
from jax.experimental.pallas import tpu as pltpu
from jax.experimental.pallas import tpu_sc as plsc

N = 10000
E = 320000
D = 128

NC = 2
NS = 16
NW = NC * NS
C = 128
NCHUNK = 80
EPW = NCHUNK * C
E_PAD = NW * EPW
NP = 10112
ZR = NP // NS
NBUF = 3

_sc_mesh = plsc.VectorSubcoreMesh(
    core_axis_name="c", subcore_axis_name="s", num_cores=NC, num_subcores=NS)


@functools.partial(
    pl.kernel,
    out_type=jax.ShapeDtypeStruct((NC, NP, D), jnp.float32),
    mesh=_sc_mesh,
    scratch_types=(
        [pltpu.VMEM((2, C), jnp.int32)] * NBUF
        + [pltpu.VMEM((C, D), jnp.float32)] * NBUF
        + [pltpu.VMEM_SHARED((NP, D), jnp.float32)]
        + [pltpu.SemaphoreType.DMA] * (2 * NBUF + 1)
    ),
)
def _sc_agg(h_hbm, pk_hbm, zero_hbm, out_hbm, *refs):
    pidx = refs[0:NBUF]
    rows = refs[NBUF:2 * NBUF]
    acc = refs[2 * NBUF]
    gsem = refs[2 * NBUF + 1:2 * NBUF + 1 + NBUF]
    isem = refs[2 * NBUF + 1 + NBUF:2 * NBUF + 1 + 2 * NBUF]
    zsem = refs[2 * NBUF + 1 + 2 * NBUF]
    cid = lax.axis_index("c")
    sid = lax.axis_index("s")
    wid = cid * NS + sid
    zcp = pltpu.async_copy(zero_hbm.at[pl.ds(sid * ZR, ZR)],
                           acc.at[pl.ds(sid * ZR, ZR)], zsem)

    def iload(j, m):
        pltpu.async_copy(pk_hbm.at[wid, j], pidx[m], isem[m])

    def iwait(j, m):
        pltpu.make_async_copy(pk_hbm.at[wid, j], pidx[m], isem[m]).wait()

    def gissue(m):
        pltpu.async_copy(h_hbm.at[pidx[m].at[0]], rows[m], gsem[m])

    def gwait(m):
        pltpu.make_async_copy(h_hbm.at[pidx[m].at[0]], rows[m], gsem[m]).wait()

    def scat(m):
        pltpu.sync_copy(rows[m], acc.at[pidx[m].at[1]], add=True)

    for j in range(NBUF):
        iload(j, j)
    for j in range(NBUF - 1):
        iwait(j, j)
        gissue(j)
    zcp.wait()
    plsc.subcore_barrier()

    def triple(k, carry):
        j = 3 * k
        for m in range(NBUF):
            gwait(m)
            scat(m)
            iload(j + m + NBUF, m)
            iwait(j + m + NBUF - 1, (m + NBUF - 1) % NBUF)
            gissue((m + NBUF - 1) % NBUF)
        return carry

    lax.fori_loop(0, (NCHUNK - 5) // 3, triple, 0)
    gwait(0)
    scat(0)
    iload(78, 0)
    iwait(77, 2)
    gissue(2)
    gwait(1)
    scat(1)
    iload(79, 1)
    iwait(78, 0)
    gissue(0)
    gwait(2)
    scat(2)
    iwait(79, 1)
    gissue(1)
    gwait(0)
    scat(0)
    gwait(1)
    scat(1)

    plsc.subcore_barrier()
    pltpu.sync_copy(acc.at[pl.ds(sid * ZR, ZR)],
                    out_hbm.at[cid, pl.ds(sid * ZR, ZR)])


@functools.partial(
    pl.kernel,
    out_type=jax.ShapeDtypeStruct((NC, NP, D), jnp.float32),
    mesh=_sc_mesh,
    scratch_types=[
        pltpu.VMEM((NCHUNK, C), jnp.int32),
        pltpu.VMEM((C, D), jnp.float32),
        pltpu.VMEM_SHARED((NP, D), jnp.float32),
        pltpu.SemaphoreType.DMA,
    ],
)
def _sc_count(dst_hbm, zero_hbm, out_hbm, didx, ones_v, acc, zsem):
    cid = lax.axis_index("c")
    sid = lax.axis_index("s")
    wid = cid * NS + sid
    zcp = pltpu.async_copy(zero_hbm.at[pl.ds(sid * ZR, ZR)],
                           acc.at[pl.ds(sid * ZR, ZR)], zsem)
    pltpu.sync_copy(dst_hbm.at[wid], didx)

    one = jnp.ones((16,), jnp.float32)

    def fill(i, carry):
        for c16 in range(D // 16):
            ones_v[i, pl.ds(c16 * 16, 16)] = one
        return carry

    lax.fori_loop(0, C, fill, 0)
    zcp.wait()
    plsc.subcore_barrier()

    def step(j, carry):
        pltpu.sync_copy(ones_v, acc.at[didx.at[j]], add=True)
        return carry

    lax.fori_loop(0, NCHUNK, step, 0)
    plsc.subcore_barrier()
    pltpu.sync_copy(acc.at[pl.ds(sid * ZR, ZR)],
                    out_hbm.at[cid, pl.ds(sid * ZR, ZR)])


_RB = 2000


def _combine_body(has_bn, h, accp, cntp, w1, w2, b, g, be, out):
    s = accp[0] + accp[1]
    c = cntp[0, :, 0:1] + cntp[1, :, 0:1]
    hn = s * (1.0 / jnp.maximum(c, 1.0))
    h2 = (lax.dot_general(h[...], w1[...], (((1,), (1,)), ((), ())),
                          preferred_element_type=jnp.float32)
          + lax.dot_general(hn, w2[...], (((1,), (1,)), ((), ())),
                            preferred_element_type=jnp.float32)
          + b[...])
    nrm = jnp.maximum(jnp.sqrt(jnp.sum(h2 * h2, axis=1, keepdims=True)), 1e-12)
    y = h2 / nrm
    if has_bn:
        y = y * (g[...] / jnp.sqrt(1.0 + 1e-5)) + be[...]
        y = jnp.maximum(y, 0.0)
    out[...] = y


def _combine(h, acc, cnt, w1, w2, b, g, be, has_bn):
    mat = pl.BlockSpec((D, D), lambda i: (0, 0))
    vec = pl.BlockSpec((1, D), lambda i: (0, 0))
    return pl.pallas_call(
        functools.partial(_combine_body, has_bn),
        grid=(N // _RB,),
        in_specs=[
            pl.BlockSpec((_RB, D), lambda i: (i, 0)),
            pl.BlockSpec((NC, _RB, D), lambda i: (0, i, 0)),
            pl.BlockSpec((NC, _RB, 8), lambda i: (0, i, 0)),
            mat, mat, vec, vec, vec,
        ],
        out_specs=pl.BlockSpec((_RB, D), lambda i: (i, 0)),
        out_shape=jax.ShapeDtypeStruct((N, D), jnp.float32),
    )(h, acc, cnt, w1, w2, b[None, :], g[None, :], be[None, :])


def kernel(x, edge_index, W1_0, W2_0, b_0, W1_1, W2_1, b_1, W1_2, W2_2, b_2,
           g_0, be_0, g_1, be_1):
    ei = edge_index.astype(jnp.int32)
    pad = E_PAD - E
    srcp = jnp.concatenate([ei[0], jnp.zeros((pad,), jnp.int32)]).reshape(
        NW, NCHUNK, C)
    dstp = jnp.concatenate([ei[1], jnp.full((pad,), NP - 1, jnp.int32)]).reshape(
        NW, NCHUNK, C)
    pk = jnp.stack([srcp, dstp], axis=2)
    zero_nd = jnp.zeros((NP, D), jnp.float32)

    cnt = _sc_count(dstp, zero_nd)[:, :, :8]

    h = x
    layers = [
        (W1_0, W2_0, b_0, g_0, be_0, True),
        (W1_1, W2_1, b_1, g_1, be_1, True),
        (W1_2, W2_2, b_2, g_1, be_1, False),
    ]
    for w1, w2, b, g, be, has_bn in layers:
        acc = _sc_agg(h, pk, zero_nd)
        h = _combine(h, acc, cnt, w1, w2, b, g, be, has_bn)
    return h


# speedup vs baseline: 3.6475x; 3.6475x over previous
import functools

import jax
import jax.numpy as jnp
from jax import lax
from jax.experimental import pallas as pl
from jax.experimental.pallas import tpu as pltpu
from jax.experimental.pallas import tpu_sc as plsc

N = 10000
E = 320000
D = 128

NC = 2
NS = 16
NW = NC * NS
C = 128
NCHUNK = 80
EPW = NCHUNK * C
E_PAD = NW * EPW
NPAIR = NCHUNK // 2
NP = 10240
ZR = NP // NS

_sc_mesh = plsc.VectorSubcoreMesh(
    core_axis_name="c", subcore_axis_name="s", num_cores=NC, num_subcores=NS)


AC = 80
ANCHUNK = 10000 // AC
AEPW = 10000


NBUF = 4


@functools.partial(
    pl.kernel,
    out_type=jax.ShapeDtypeStruct((NC, NP, D), jnp.float32),
    mesh=_sc_mesh,
    scratch_types=(
        [pltpu.VMEM((AC,), jnp.int32)] * NBUF
        + [pltpu.VMEM((AC,), jnp.int32)] * NBUF
        + [pltpu.VMEM((AC, D), jnp.float32)] * NBUF
        + [pltpu.VMEM_SHARED((NP, D), jnp.float32)]
        + [pltpu.SemaphoreType.DMA] * (2 * NBUF + 1)
    ),
)
def _sc_agg(h_hbm, src_hbm, dst_hbm, zero_hbm, out_hbm, *refs):
    sidx = refs[0:NBUF]
    didx = refs[NBUF:2 * NBUF]
    rows = refs[2 * NBUF:3 * NBUF]
    acc = refs[3 * NBUF]
    gsem = refs[3 * NBUF + 1:3 * NBUF + 1 + NBUF]
    isem = refs[3 * NBUF + 1 + NBUF:3 * NBUF + 1 + 2 * NBUF]
    zsem = refs[3 * NBUF + 1 + 2 * NBUF]
    cid = lax.axis_index("c")
    sid = lax.axis_index("s")
    wid = cid * NS + sid
    zcp = pltpu.async_copy(zero_hbm.at[pl.ds(sid * ZR, ZR)],
                           acc.at[pl.ds(sid * ZR, ZR)], zsem)

    def iload(j, m):
        e0 = wid * AEPW + j * AC
        pltpu.async_copy(src_hbm.at[pl.ds(e0, AC)], sidx[m], isem[m])
        pltpu.async_copy(dst_hbm.at[pl.ds(e0, AC)], didx[m], isem[m])

    def iwait(j, m):
        e0 = wid * AEPW + j * AC
        pltpu.make_async_copy(src_hbm.at[pl.ds(e0, AC)], sidx[m], isem[m]).wait()
        pltpu.make_async_copy(dst_hbm.at[pl.ds(e0, AC)], didx[m], isem[m]).wait()

    def gissue(m):
        pltpu.async_copy(h_hbm.at[sidx[m]], rows[m], gsem[m])

    def gwait(m):
        pltpu.make_async_copy(h_hbm.at[sidx[m]], rows[m], gsem[m]).wait()

    def scat(m):
        pltpu.sync_copy(rows[m], acc.at[didx[m]], add=True)

    for j in range(NBUF):
        iload(j, j)
    for j in range(NBUF - 1):
        iwait(j, j)
        gissue(j)
    zcp.wait()
    plsc.subcore_barrier()

    def quad(k, carry):
        j = 4 * k
        for m in range(4):
            gwait(m)
            scat(m)
            iload(j + m + NBUF, m)
            iwait(j + m + NBUF - 1, (m + NBUF - 1) % NBUF)
            gissue((m + NBUF - 1) % NBUF)
        return carry

    lax.fori_loop(0, (ANCHUNK - 5) // 4, quad, 0)
    gwait(0); scat(0)
    iload(124, 0)
    iwait(123, 3)
    gissue(3)
    gwait(1); scat(1)
    iwait(124, 0)
    gissue(0)
    gwait(2); scat(2)
    gwait(3); scat(3)
    gwait(0); scat(0)

    plsc.subcore_barrier()
    pltpu.sync_copy(acc.at[pl.ds(sid * ZR, ZR)],
                    out_hbm.at[cid, pl.ds(sid * ZR, ZR)])


@functools.partial(
    pl.kernel,
    out_type=jax.ShapeDtypeStruct((NC, NP, D), jnp.float32),
    mesh=_sc_mesh,
    scratch_types=[
        pltpu.VMEM((NCHUNK, C), jnp.int32),
        pltpu.VMEM((C, D), jnp.float32),
        pltpu.VMEM_SHARED((NP, D), jnp.float32),
        pltpu.SemaphoreType.DMA,
    ],
)
def _sc_count(dst_hbm, zero_hbm, out_hbm, didx, ones_v, acc, zsem):
    cid = lax.axis_index("c")
    sid = lax.axis_index("s")
    wid = cid * NS + sid
    zcp = pltpu.async_copy(zero_hbm.at[pl.ds(sid * ZR, ZR)],
                           acc.at[pl.ds(sid * ZR, ZR)], zsem)
    pltpu.sync_copy(dst_hbm.at[wid], didx)

    one = jnp.ones((16,), jnp.float32)

    def fill(i, carry):
        for c16 in range(D // 16):
            ones_v[i, pl.ds(c16 * 16, 16)] = one
        return carry

    lax.fori_loop(0, C, fill, 0)
    zcp.wait()
    plsc.subcore_barrier()

    def step(j, carry):
        pltpu.sync_copy(ones_v, acc.at[didx.at[j]], add=True)
        return carry

    lax.fori_loop(0, NCHUNK, step, 0)
    plsc.subcore_barrier()
    pltpu.sync_copy(acc.at[pl.ds(sid * ZR, ZR)],
                    out_hbm.at[cid, pl.ds(sid * ZR, ZR)])


_RB = 2000


def _combine_body(has_bn, h, accp, cntp, w1, w2, b, g, be, out):
    s = accp[0] + accp[1]
    c = cntp[0, :, 0:1] + cntp[1, :, 0:1]
    hn = s * (1.0 / jnp.maximum(c, 1.0))
    h2 = (lax.dot_general(h[...], w1[...], (((1,), (1,)), ((), ())),
                          preferred_element_type=jnp.float32)
          + lax.dot_general(hn, w2[...], (((1,), (1,)), ((), ())),
                            preferred_element_type=jnp.float32)
          + b[...])
    nrm = jnp.maximum(jnp.sqrt(jnp.sum(h2 * h2, axis=1, keepdims=True)), 1e-12)
    y = h2 / nrm
    if has_bn:
        y = y * (g[...] / jnp.sqrt(1.0 + 1e-5)) + be[...]
        y = jnp.maximum(y, 0.0)
    out[...] = y


def _combine(h, acc, cnt, w1, w2, b, g, be, has_bn):
    mat = pl.BlockSpec((D, D), lambda i: (0, 0))
    vec = pl.BlockSpec((1, D), lambda i: (0, 0))
    return pl.pallas_call(
        functools.partial(_combine_body, has_bn),
        grid=(N // _RB,),
        in_specs=[
            pl.BlockSpec((_RB, D), lambda i: (i, 0)),
            pl.BlockSpec((NC, _RB, D), lambda i: (0, i, 0)),
            pl.BlockSpec((NC, _RB, 8), lambda i: (0, i, 0)),
            mat, mat, vec, vec, vec,
        ],
        out_specs=pl.BlockSpec((_RB, D), lambda i: (i, 0)),
        out_shape=jax.ShapeDtypeStruct((N, D), jnp.float32),
    )(h, acc, cnt, w1, w2, b[None, :], g[None, :], be[None, :])


def kernel(x, edge_index, W1_0, W2_0, b_0, W1_1, W2_1, b_1, W1_2, W2_2, b_2,
           g_0, be_0, g_1, be_1):
    ei = edge_index.astype(jnp.int32)
    pad = E_PAD - E
    src = ei[0]
    dst = ei[1]
    dst3 = jnp.concatenate([dst, jnp.full((pad,), NP - 1, jnp.int32)]).reshape(
        NW, NCHUNK, C)
    zero_nd = jnp.zeros((NP, D), jnp.float32)

    cnt = _sc_count(dst3, zero_nd)[:, :, :8]

    h = x
    layers = [
        (W1_0, W2_0, b_0, g_0, be_0, True),
        (W1_1, W2_1, b_1, g_1, be_1, True),
        (W1_2, W2_2, b_2, g_1, be_1, False),
    ]
    for w1, w2, b, g, be, has_bn in layers:
        acc = _sc_agg(h, src, dst, zero_nd)
        h = _combine(h, acc, cnt, w1, w2, b, g, be, has_bn)
    return h


# speedup vs baseline: 3.7320x; 1.0232x over previous
import functools

import jax
import jax.numpy as jnp
from jax import lax
from jax.experimental import pallas as pl
from jax.experimental.pallas import tpu as pltpu
from jax.experimental.pallas import tpu_sc as plsc

N = 10000
E = 320000
D = 128

NC = 2
NS = 16
NW = NC * NS
C = 128
NCHUNK = 80
EPW = NCHUNK * C
E_PAD = NW * EPW
NPAIR = NCHUNK // 2
NP = 10240
ZR = NP // NS

_sc_mesh = plsc.VectorSubcoreMesh(
    core_axis_name="c", subcore_axis_name="s", num_cores=NC, num_subcores=NS)


AC = 80
ANCHUNK = 10000 // AC
AEPW = 10000


NBUF = 4


@functools.partial(
    pl.kernel,
    out_type=jax.ShapeDtypeStruct((NC, NP, D), jnp.float32),
    mesh=_sc_mesh,
    scratch_types=(
        [pltpu.VMEM((AC,), jnp.int32)] * NBUF
        + [pltpu.VMEM((AC,), jnp.int32)] * NBUF
        + [pltpu.VMEM((AC, D), jnp.float32)] * NBUF
        + [pltpu.VMEM_SHARED((NP, D), jnp.float32)]
        + [pltpu.SemaphoreType.DMA] * (2 * NBUF + 1)
    ),
)
def _sc_agg(h_hbm, src_hbm, dst_hbm, zero_hbm, out_hbm, *refs):
    sidx = refs[0:NBUF]
    didx = refs[NBUF:2 * NBUF]
    rows = refs[2 * NBUF:3 * NBUF]
    acc = refs[3 * NBUF]
    gsem = refs[3 * NBUF + 1:3 * NBUF + 1 + NBUF]
    isem = refs[3 * NBUF + 1 + NBUF:3 * NBUF + 1 + 2 * NBUF]
    zsem = refs[3 * NBUF + 1 + 2 * NBUF]
    cid = lax.axis_index("c")
    sid = lax.axis_index("s")
    wid = cid * NS + sid
    zcp = pltpu.async_copy(zero_hbm.at[pl.ds(sid * ZR, ZR)],
                           acc.at[pl.ds(sid * ZR, ZR)], zsem)

    def iload(j, m):
        e0 = wid * AEPW + j * AC
        pltpu.async_copy(src_hbm.at[pl.ds(e0, AC)], sidx[m], isem[m])
        pltpu.async_copy(dst_hbm.at[pl.ds(e0, AC)], didx[m], isem[m])

    def iwait(j, m):
        e0 = wid * AEPW + j * AC
        pltpu.make_async_copy(src_hbm.at[pl.ds(e0, AC)], sidx[m], isem[m]).wait()
        pltpu.make_async_copy(dst_hbm.at[pl.ds(e0, AC)], didx[m], isem[m]).wait()

    def gissue(m):
        pltpu.async_copy(h_hbm.at[sidx[m]], rows[m], gsem[m])

    def gwait(m):
        pltpu.make_async_copy(h_hbm.at[sidx[m]], rows[m], gsem[m]).wait()

    def scat(m):
        pltpu.sync_copy(rows[m], acc.at[didx[m]], add=True)

    for j in range(NBUF):
        iload(j, j)
    for j in range(NBUF - 1):
        iwait(j, j)
        gissue(j)
    zcp.wait()
    plsc.subcore_barrier()

    def quad(k, carry):
        j = 4 * k
        for m in range(4):
            gwait(m)
            scat(m)
            iload(j + m + NBUF, m)
            iwait(j + m + NBUF - 1, (m + NBUF - 1) % NBUF)
            gissue((m + NBUF - 1) % NBUF)
        return carry

    lax.fori_loop(0, (ANCHUNK - 5) // 4, quad, 0)
    gwait(0); scat(0)
    iload(124, 0)
    iwait(123, 3)
    gissue(3)
    gwait(1); scat(1)
    iwait(124, 0)
    gissue(0)
    gwait(2); scat(2)
    gwait(3); scat(3)
    gwait(0); scat(0)

    plsc.subcore_barrier()
    pltpu.sync_copy(acc.at[pl.ds(sid * ZR, ZR)],
                    out_hbm.at[cid, pl.ds(sid * ZR, ZR)])


@functools.partial(
    pl.kernel,
    out_type=(jax.ShapeDtypeStruct((NC, NP, D), jnp.float32),
              jax.ShapeDtypeStruct((NC, NP, D), jnp.float32)),
    mesh=_sc_mesh,
    scratch_types=(
        [pltpu.VMEM((AC,), jnp.int32)] * NBUF
        + [pltpu.VMEM((AC,), jnp.int32)] * NBUF
        + [pltpu.VMEM((AC, D), jnp.float32)] * NBUF
        + [pltpu.VMEM_SHARED((NP, D), jnp.float32)]
        + [pltpu.SemaphoreType.DMA] * (2 * NBUF + 1)
    ),
)
def _sc_agg_count(h_hbm, src_hbm, dst_hbm, zero_hbm, cnt_hbm, out_hbm, *refs):
    sidx = refs[0:NBUF]
    didx = refs[NBUF:2 * NBUF]
    rows = refs[2 * NBUF:3 * NBUF]
    acc = refs[3 * NBUF]
    gsem = refs[3 * NBUF + 1:3 * NBUF + 1 + NBUF]
    isem = refs[3 * NBUF + 1 + NBUF:3 * NBUF + 1 + 2 * NBUF]
    zsem = refs[3 * NBUF + 1 + 2 * NBUF]
    cid = lax.axis_index("c")
    sid = lax.axis_index("s")
    wid = cid * NS + sid

    zcp = pltpu.async_copy(zero_hbm.at[pl.ds(sid * ZR, ZR)],
                           acc.at[pl.ds(sid * ZR, ZR)], zsem)
    ones_v = rows[NBUF - 1]
    one = jnp.ones((16,), jnp.float32)

    def fill(i, carry):
        for c16 in range(D // 16):
            ones_v[i, pl.ds(c16 * 16, 16)] = one
        return carry

    lax.fori_loop(0, AC, fill, 0)

    def dload(j, m):
        e0 = wid * AEPW + j * AC
        pltpu.async_copy(dst_hbm.at[pl.ds(e0, AC)], didx[m], isem[m])

    def dwait(j, m):
        e0 = wid * AEPW + j * AC
        pltpu.make_async_copy(dst_hbm.at[pl.ds(e0, AC)], didx[m], isem[m]).wait()

    def scat1(m):
        pltpu.sync_copy(ones_v, acc.at[didx[m]], add=True)

    for j in range(NBUF):
        dload(j, j)
    zcp.wait()
    plsc.subcore_barrier()

    def cquad(k, carry):
        j = 4 * k
        for m in range(4):
            dwait(j + m, m)
            scat1(m)
            dload(j + m + NBUF, m)
        return carry

    lax.fori_loop(0, (ANCHUNK - 5) // 4, cquad, 0)
    dwait(120, 0); scat1(0); dload(124, 0)
    dwait(121, 1); scat1(1)
    dwait(122, 2); scat1(2)
    dwait(123, 3); scat1(3)
    dwait(124, 0); scat1(0)
    plsc.subcore_barrier()
    pltpu.sync_copy(acc.at[pl.ds(sid * ZR, ZR)],
                    cnt_hbm.at[cid, pl.ds(sid * ZR, ZR)])

    zcp2 = pltpu.async_copy(zero_hbm.at[pl.ds(sid * ZR, ZR)],
                            acc.at[pl.ds(sid * ZR, ZR)], zsem)

    def iload(j, m):
        e0 = wid * AEPW + j * AC
        pltpu.async_copy(src_hbm.at[pl.ds(e0, AC)], sidx[m], isem[m])
        pltpu.async_copy(dst_hbm.at[pl.ds(e0, AC)], didx[m], isem[m])

    def iwait(j, m):
        e0 = wid * AEPW + j * AC
        pltpu.make_async_copy(src_hbm.at[pl.ds(e0, AC)], sidx[m], isem[m]).wait()
        pltpu.make_async_copy(dst_hbm.at[pl.ds(e0, AC)], didx[m], isem[m]).wait()

    def gissue(m):
        pltpu.async_copy(h_hbm.at[sidx[m]], rows[m], gsem[m])

    def gwait(m):
        pltpu.make_async_copy(h_hbm.at[sidx[m]], rows[m], gsem[m]).wait()

    def scat(m):
        pltpu.sync_copy(rows[m], acc.at[didx[m]], add=True)

    for j in range(NBUF):
        iload(j, j)
    for j in range(NBUF - 1):
        iwait(j, j)
        gissue(j)
    zcp2.wait()
    plsc.subcore_barrier()

    def quad(k, carry):
        j = 4 * k
        for m in range(4):
            gwait(m)
            scat(m)
            iload(j + m + NBUF, m)
            iwait(j + m + NBUF - 1, (m + NBUF - 1) % NBUF)
            gissue((m + NBUF - 1) % NBUF)
        return carry

    lax.fori_loop(0, (ANCHUNK - 5) // 4, quad, 0)
    gwait(0); scat(0)
    iload(124, 0)
    iwait(123, 3)
    gissue(3)
    gwait(1); scat(1)
    iwait(124, 0)
    gissue(0)
    gwait(2); scat(2)
    gwait(3); scat(3)
    gwait(0); scat(0)

    plsc.subcore_barrier()
    pltpu.sync_copy(acc.at[pl.ds(sid * ZR, ZR)],
                    out_hbm.at[cid, pl.ds(sid * ZR, ZR)])


_RB = 2000


def _combine_body(has_bn, h, accp, cntp, w1, w2, b, g, be, out):
    s = accp[0] + accp[1]
    c = cntp[0, :, 0:1] + cntp[1, :, 0:1]
    hn = s * (1.0 / jnp.maximum(c, 1.0))
    h2 = (lax.dot_general(h[...], w1[...], (((1,), (1,)), ((), ())),
                          preferred_element_type=jnp.float32)
          + lax.dot_general(hn, w2[...], (((1,), (1,)), ((), ())),
                            preferred_element_type=jnp.float32)
          + b[...])
    nrm = jnp.maximum(jnp.sqrt(jnp.sum(h2 * h2, axis=1, keepdims=True)), 1e-12)
    y = h2 / nrm
    if has_bn:
        y = y * (g[...] / jnp.sqrt(1.0 + 1e-5)) + be[...]
        y = jnp.maximum(y, 0.0)
    out[...] = y


def _combine(h, acc, cnt, w1, w2, b, g, be, has_bn):
    mat = pl.BlockSpec((D, D), lambda i: (0, 0))
    vec = pl.BlockSpec((1, D), lambda i: (0, 0))
    return pl.pallas_call(
        functools.partial(_combine_body, has_bn),
        grid=(N // _RB,),
        in_specs=[
            pl.BlockSpec((_RB, D), lambda i: (i, 0)),
            pl.BlockSpec((NC, _RB, D), lambda i: (0, i, 0)),
            pl.BlockSpec((NC, _RB, 8), lambda i: (0, i, 0)),
            mat, mat, vec, vec, vec,
        ],
        out_specs=pl.BlockSpec((_RB, D), lambda i: (i, 0)),
        out_shape=jax.ShapeDtypeStruct((N, D), jnp.float32),
    )(h, acc, cnt, w1, w2, b[None, :], g[None, :], be[None, :])


def kernel(x, edge_index, W1_0, W2_0, b_0, W1_1, W2_1, b_1, W1_2, W2_2, b_2,
           g_0, be_0, g_1, be_1):
    ei = edge_index.astype(jnp.int32)
    src = ei[0]
    dst = ei[1]
    zero_nd = jnp.zeros((NP, D), jnp.float32)

    cnt_full, acc = _sc_agg_count(x, src, dst, zero_nd)
    cnt = cnt_full[:, :, :8]
    h = _combine(x, acc, cnt, W1_0, W2_0, b_0, g_0, be_0, True)
    for w1, w2, b, g, be, has_bn in [
        (W1_1, W2_1, b_1, g_1, be_1, True),
        (W1_2, W2_2, b_2, g_1, be_1, False),
    ]:
        acc = _sc_agg(h, src, dst, zero_nd)
        h = _combine(h, acc, cnt, w1, w2, b, g, be, has_bn)
    return h


# speedup vs baseline: 3.7536x; 1.0058x over previous
import functools

import jax
import jax.numpy as jnp
from jax import lax
from jax.experimental import pallas as pl
from jax.experimental.pallas import tpu as pltpu
from jax.experimental.pallas import tpu_sc as plsc

N = 10000
E = 320000
D = 128

NC = 2
NS = 16
NW = NC * NS
C = 128
NCHUNK = 80
EPW = NCHUNK * C
E_PAD = NW * EPW
NPAIR = NCHUNK // 2
NP = 10240
ZR = NP // NS

_sc_mesh = plsc.VectorSubcoreMesh(
    core_axis_name="c", subcore_axis_name="s", num_cores=NC, num_subcores=NS)


AC = 80
ANCHUNK = 10000 // AC
AEPW = 10000


NBUF = 4


@functools.partial(
    pl.kernel,
    out_type=jax.ShapeDtypeStruct((NC, NP, D), jnp.float32),
    mesh=_sc_mesh,
    scratch_types=(
        [pltpu.VMEM((AC,), jnp.int32)] * NBUF
        + [pltpu.VMEM((AC,), jnp.int32)] * NBUF
        + [pltpu.VMEM((AC, D), jnp.float32)] * NBUF
        + [pltpu.VMEM_SHARED((NP, D), jnp.float32)]
        + [pltpu.SemaphoreType.DMA] * (2 * NBUF + 1)
    ),
)
def _sc_agg(h_hbm, src_hbm, dst_hbm, zero_hbm, out_hbm, *refs):
    sidx = refs[0:NBUF]
    didx = refs[NBUF:2 * NBUF]
    rows = refs[2 * NBUF:3 * NBUF]
    acc = refs[3 * NBUF]
    gsem = refs[3 * NBUF + 1:3 * NBUF + 1 + NBUF]
    isem = refs[3 * NBUF + 1 + NBUF:3 * NBUF + 1 + 2 * NBUF]
    zsem = refs[3 * NBUF + 1 + 2 * NBUF]
    cid = lax.axis_index("c")
    sid = lax.axis_index("s")
    wid = cid * NS + sid
    zcp = pltpu.async_copy(zero_hbm.at[pl.ds(sid * ZR, ZR)],
                           acc.at[pl.ds(sid * ZR, ZR)], zsem)

    def iload(j, m):
        e0 = wid * AEPW + j * AC
        pltpu.async_copy(src_hbm.at[pl.ds(e0, AC)], sidx[m], isem[m])
        pltpu.async_copy(dst_hbm.at[pl.ds(e0, AC)], didx[m], isem[m])

    def iwait(j, m):
        e0 = wid * AEPW + j * AC
        pltpu.make_async_copy(src_hbm.at[pl.ds(e0, AC)], sidx[m], isem[m]).wait()
        pltpu.make_async_copy(dst_hbm.at[pl.ds(e0, AC)], didx[m], isem[m]).wait()

    def gissue(m):
        pltpu.async_copy(h_hbm.at[sidx[m]], rows[m], gsem[m])

    def gwait(m):
        pltpu.make_async_copy(h_hbm.at[sidx[m]], rows[m], gsem[m]).wait()

    def scat(m):
        pltpu.sync_copy(rows[m], acc.at[didx[m]], add=True)

    for j in range(NBUF):
        iload(j, j)
    for j in range(NBUF - 1):
        iwait(j, j)
        gissue(j)
    zcp.wait()
    plsc.subcore_barrier()

    def quad(k, carry):
        j = 4 * k
        for m in range(4):
            gwait(m)
            scat(m)
            iload(j + m + NBUF, m)
            iwait(j + m + NBUF - 1, (m + NBUF - 1) % NBUF)
            gissue((m + NBUF - 1) % NBUF)
        return carry

    lax.fori_loop(0, (ANCHUNK - 5) // 4, quad, 0)
    gwait(0); scat(0)
    iload(124, 0)
    iwait(123, 3)
    gissue(3)
    gwait(1); scat(1)
    iwait(124, 0)
    gissue(0)
    gwait(2); scat(2)
    gwait(3); scat(3)
    gwait(0); scat(0)

    plsc.subcore_barrier()
    pltpu.sync_copy(acc.at[pl.ds(sid * ZR, ZR)],
                    out_hbm.at[cid, pl.ds(sid * ZR, ZR)])


@functools.partial(
    pl.kernel,
    out_type=(jax.ShapeDtypeStruct((NC, NP, D), jnp.float32),
              jax.ShapeDtypeStruct((NC, NP, D), jnp.float32)),
    mesh=_sc_mesh,
    scratch_types=(
        [pltpu.VMEM((AC,), jnp.int32)] * NBUF
        + [pltpu.VMEM((AC,), jnp.int32)] * NBUF
        + [pltpu.VMEM((AC, D), jnp.float32)] * NBUF
        + [pltpu.VMEM_SHARED((NP, D), jnp.float32)]
        + [pltpu.SemaphoreType.DMA] * (2 * NBUF + 1)
    ),
)
def _sc_agg_count(h_hbm, src_hbm, dst_hbm, zero_hbm, cnt_hbm, out_hbm, *refs):
    sidx = refs[0:NBUF]
    didx = refs[NBUF:2 * NBUF]
    rows = refs[2 * NBUF:3 * NBUF]
    acc = refs[3 * NBUF]
    gsem = refs[3 * NBUF + 1:3 * NBUF + 1 + NBUF]
    isem = refs[3 * NBUF + 1 + NBUF:3 * NBUF + 1 + 2 * NBUF]
    zsem = refs[3 * NBUF + 1 + 2 * NBUF]
    cid = lax.axis_index("c")
    sid = lax.axis_index("s")
    wid = cid * NS + sid

    zcp = pltpu.async_copy(zero_hbm.at[pl.ds(sid * ZR, ZR)],
                           acc.at[pl.ds(sid * ZR, ZR)], zsem)
    ones_v = rows[NBUF - 1]
    one = jnp.ones((16,), jnp.float32)

    def fill(i, carry):
        for c16 in range(D // 16):
            ones_v[i, pl.ds(c16 * 16, 16)] = one
        return carry

    lax.fori_loop(0, AC, fill, 0)

    def dload(j, m):
        e0 = wid * AEPW + j * AC
        pltpu.async_copy(dst_hbm.at[pl.ds(e0, AC)], didx[m], isem[m])

    def dwait(j, m):
        e0 = wid * AEPW + j * AC
        pltpu.make_async_copy(dst_hbm.at[pl.ds(e0, AC)], didx[m], isem[m]).wait()

    def scat1(m):
        pltpu.sync_copy(ones_v, acc.at[didx[m]], add=True)

    for j in range(NBUF):
        dload(j, j)
    zcp.wait()
    plsc.subcore_barrier()

    def cquad(k, carry):
        j = 4 * k
        for m in range(4):
            dwait(j + m, m)
            scat1(m)
            dload(j + m + NBUF, m)
        return carry

    lax.fori_loop(0, (ANCHUNK - 5) // 4, cquad, 0)
    dwait(120, 0); scat1(0); dload(124, 0)
    dwait(121, 1); scat1(1)
    dwait(122, 2); scat1(2)
    dwait(123, 3); scat1(3)
    dwait(124, 0); scat1(0)
    plsc.subcore_barrier()
    pltpu.sync_copy(acc.at[pl.ds(sid * ZR, ZR)],
                    cnt_hbm.at[cid, pl.ds(sid * ZR, ZR)])

    zcp2 = pltpu.async_copy(zero_hbm.at[pl.ds(sid * ZR, ZR)],
                            acc.at[pl.ds(sid * ZR, ZR)], zsem)

    def iload(j, m):
        e0 = wid * AEPW + j * AC
        pltpu.async_copy(src_hbm.at[pl.ds(e0, AC)], sidx[m], isem[m])
        pltpu.async_copy(dst_hbm.at[pl.ds(e0, AC)], didx[m], isem[m])

    def iwait(j, m):
        e0 = wid * AEPW + j * AC
        pltpu.make_async_copy(src_hbm.at[pl.ds(e0, AC)], sidx[m], isem[m]).wait()
        pltpu.make_async_copy(dst_hbm.at[pl.ds(e0, AC)], didx[m], isem[m]).wait()

    def gissue(m):
        pltpu.async_copy(h_hbm.at[sidx[m]], rows[m], gsem[m])

    def gwait(m):
        pltpu.make_async_copy(h_hbm.at[sidx[m]], rows[m], gsem[m]).wait()

    def scat(m):
        pltpu.sync_copy(rows[m], acc.at[didx[m]], add=True)

    for j in range(NBUF):
        iload(j, j)
    for j in range(NBUF - 1):
        iwait(j, j)
        gissue(j)
    zcp2.wait()
    plsc.subcore_barrier()

    def quad(k, carry):
        j = 4 * k
        for m in range(4):
            gwait(m)
            scat(m)
            iload(j + m + NBUF, m)
            iwait(j + m + NBUF - 1, (m + NBUF - 1) % NBUF)
            gissue((m + NBUF - 1) % NBUF)
        return carry

    lax.fori_loop(0, (ANCHUNK - 5) // 4, quad, 0)
    gwait(0); scat(0)
    iload(124, 0)
    iwait(123, 3)
    gissue(3)
    gwait(1); scat(1)
    iwait(124, 0)
    gissue(0)
    gwait(2); scat(2)
    gwait(3); scat(3)
    gwait(0); scat(0)

    plsc.subcore_barrier()
    pltpu.sync_copy(acc.at[pl.ds(sid * ZR, ZR)],
                    out_hbm.at[cid, pl.ds(sid * ZR, ZR)])


_RB = 5000


def _combine_body(has_bn, h, accp, cntp, w1, w2, b, g, be, out):
    s = accp[0] + accp[1]
    c = cntp[0, :, 0:1] + cntp[1, :, 0:1]
    hn = s * (1.0 / jnp.maximum(c, 1.0))
    h2 = (lax.dot_general(h[...], w1[...], (((1,), (1,)), ((), ())),
                          preferred_element_type=jnp.float32)
          + lax.dot_general(hn, w2[...], (((1,), (1,)), ((), ())),
                            preferred_element_type=jnp.float32)
          + b[...])
    nrm = jnp.maximum(jnp.sqrt(jnp.sum(h2 * h2, axis=1, keepdims=True)), 1e-12)
    y = h2 / nrm
    if has_bn:
        y = y * (g[...] / jnp.sqrt(1.0 + 1e-5)) + be[...]
        y = jnp.maximum(y, 0.0)
    out[...] = y


def _combine(h, acc, cnt, w1, w2, b, g, be, has_bn):
    mat = pl.BlockSpec((D, D), lambda i: (0, 0))
    vec = pl.BlockSpec((1, D), lambda i: (0, 0))
    return pl.pallas_call(
        functools.partial(_combine_body, has_bn),
        grid=(N // _RB,),
        in_specs=[
            pl.BlockSpec((_RB, D), lambda i: (i, 0)),
            pl.BlockSpec((NC, _RB, D), lambda i: (0, i, 0)),
            pl.BlockSpec((NC, _RB, 8), lambda i: (0, i, 0)),
            mat, mat, vec, vec, vec,
        ],
        out_specs=pl.BlockSpec((_RB, D), lambda i: (i, 0)),
        out_shape=jax.ShapeDtypeStruct((N, D), jnp.float32),
    )(h, acc, cnt, w1, w2, b[None, :], g[None, :], be[None, :])


def kernel(x, edge_index, W1_0, W2_0, b_0, W1_1, W2_1, b_1, W1_2, W2_2, b_2,
           g_0, be_0, g_1, be_1):
    ei = edge_index.astype(jnp.int32)
    src = ei[0]
    dst = ei[1]
    zero_nd = jnp.zeros((NP, D), jnp.float32)

    cnt_full, acc = _sc_agg_count(x, src, dst, zero_nd)
    cnt = cnt_full[:, :, :8]
    h = _combine(x, acc, cnt, W1_0, W2_0, b_0, g_0, be_0, True)
    for w1, w2, b, g, be, has_bn in [
        (W1_1, W2_1, b_1, g_1, be_1, True),
        (W1_2, W2_2, b_2, g_1, be_1, False),
    ]:
        acc = _sc_agg(h, src, dst, zero_nd)
        h = _combine(h, acc, cnt, w1, w2, b, g, be, has_bn)
    return h


# speedup vs baseline: 3.7553x; 1.0005x over previous
import functools

import jax
import jax.numpy as jnp
from jax import lax
from jax.experimental import pallas as pl
from jax.experimental.pallas import tpu as pltpu
from jax.experimental.pallas import tpu_sc as plsc

N = 10000
E = 320000
D = 128

NC = 2
NS = 16
NW = NC * NS
NP = 10240
ZR = NP // NS

_sc_mesh = plsc.VectorSubcoreMesh(
    core_axis_name="c", subcore_axis_name="s", num_cores=NC, num_subcores=NS)


AC = 80
ANCHUNK = 10000 // AC
AEPW = 10000


NBUF = 4


@functools.partial(
    pl.kernel,
    out_type=jax.ShapeDtypeStruct((NC, NP, D), jnp.float32),
    mesh=_sc_mesh,
    scratch_types=(
        [pltpu.VMEM((AC,), jnp.int32)] * NBUF
        + [pltpu.VMEM((AC,), jnp.int32)] * NBUF
        + [pltpu.VMEM((AC, D), jnp.float32)] * NBUF
        + [pltpu.VMEM_SHARED((NP, D), jnp.float32)]
        + [pltpu.SemaphoreType.DMA] * (2 * NBUF + 1)
    ),
)
def _sc_agg(h_hbm, src_hbm, dst_hbm, zero_hbm, out_hbm, *refs):
    sidx = refs[0:NBUF]
    didx = refs[NBUF:2 * NBUF]
    rows = refs[2 * NBUF:3 * NBUF]
    acc = refs[3 * NBUF]
    gsem = refs[3 * NBUF + 1:3 * NBUF + 1 + NBUF]
    isem = refs[3 * NBUF + 1 + NBUF:3 * NBUF + 1 + 2 * NBUF]
    zsem = refs[3 * NBUF + 1 + 2 * NBUF]
    cid = lax.axis_index("c")
    sid = lax.axis_index("s")
    wid = cid * NS + sid
    zcp = pltpu.async_copy(zero_hbm.at[pl.ds(sid * ZR, ZR)],
                           acc.at[pl.ds(sid * ZR, ZR)], zsem)

    def iload(j, m):
        e0 = wid * AEPW + j * AC
        pltpu.async_copy(src_hbm.at[pl.ds(e0, AC)], sidx[m], isem[m])
        pltpu.async_copy(dst_hbm.at[pl.ds(e0, AC)], didx[m], isem[m])

    def iwait(j, m):
        e0 = wid * AEPW + j * AC
        pltpu.make_async_copy(src_hbm.at[pl.ds(e0, AC)], sidx[m], isem[m]).wait()
        pltpu.make_async_copy(dst_hbm.at[pl.ds(e0, AC)], didx[m], isem[m]).wait()

    def gissue(m):
        pltpu.async_copy(h_hbm.at[sidx[m]], rows[m], gsem[m])

    def gwait(m):
        pltpu.make_async_copy(h_hbm.at[sidx[m]], rows[m], gsem[m]).wait()

    def scat(m):
        pltpu.sync_copy(rows[m], acc.at[didx[m]], add=True)

    for j in range(NBUF):
        iload(j, j)
    for j in range(NBUF - 1):
        iwait(j, j)
        gissue(j)
    zcp.wait()
    plsc.subcore_barrier()

    def quad(k, carry):
        j = 4 * k
        for m in range(4):
            gwait(m)
            scat(m)
            iload(j + m + NBUF, m)
            iwait(j + m + NBUF - 1, (m + NBUF - 1) % NBUF)
            gissue((m + NBUF - 1) % NBUF)
        return carry

    lax.fori_loop(0, (ANCHUNK - 5) // 4, quad, 0)
    gwait(0); scat(0)
    iload(124, 0)
    iwait(123, 3)
    gissue(3)
    gwait(1); scat(1)
    iwait(124, 0)
    gissue(0)
    gwait(2); scat(2)
    gwait(3); scat(3)
    gwait(0); scat(0)

    plsc.subcore_barrier()
    pltpu.sync_copy(acc.at[pl.ds(sid * ZR, ZR)],
                    out_hbm.at[cid, pl.ds(sid * ZR, ZR)])


@functools.partial(
    pl.kernel,
    out_type=(jax.ShapeDtypeStruct((NC, NP, D), jnp.float32),
              jax.ShapeDtypeStruct((NC, NP, D), jnp.float32)),
    mesh=_sc_mesh,
    scratch_types=(
        [pltpu.VMEM((AC,), jnp.int32)] * NBUF
        + [pltpu.VMEM((AC,), jnp.int32)] * NBUF
        + [pltpu.VMEM((AC, D), jnp.float32)] * NBUF
        + [pltpu.VMEM_SHARED((NP, D), jnp.float32)]
        + [pltpu.SemaphoreType.DMA] * (2 * NBUF + 1)
    ),
)
def _sc_agg_count(h_hbm, src_hbm, dst_hbm, zero_hbm, cnt_hbm, out_hbm, *refs):
    sidx = refs[0:NBUF]
    didx = refs[NBUF:2 * NBUF]
    rows = refs[2 * NBUF:3 * NBUF]
    acc = refs[3 * NBUF]
    gsem = refs[3 * NBUF + 1:3 * NBUF + 1 + NBUF]
    isem = refs[3 * NBUF + 1 + NBUF:3 * NBUF + 1 + 2 * NBUF]
    zsem = refs[3 * NBUF + 1 + 2 * NBUF]
    cid = lax.axis_index("c")
    sid = lax.axis_index("s")
    wid = cid * NS + sid

    zcp = pltpu.async_copy(zero_hbm.at[pl.ds(sid * ZR, ZR)],
                           acc.at[pl.ds(sid * ZR, ZR)], zsem)
    ones_v = rows[NBUF - 1]
    one = jnp.ones((16,), jnp.float32)

    def fill(i, carry):
        for c16 in range(D // 16):
            ones_v[i, pl.ds(c16 * 16, 16)] = one
        return carry

    lax.fori_loop(0, AC, fill, 0)

    def dload(j, m):
        e0 = wid * AEPW + j * AC
        pltpu.async_copy(dst_hbm.at[pl.ds(e0, AC)], didx[m], isem[m])

    def dwait(j, m):
        e0 = wid * AEPW + j * AC
        pltpu.make_async_copy(dst_hbm.at[pl.ds(e0, AC)], didx[m], isem[m]).wait()

    def scat1(m):
        pltpu.sync_copy(ones_v, acc.at[didx[m]], add=True)

    for j in range(NBUF):
        dload(j, j)
    zcp.wait()
    plsc.subcore_barrier()

    def cquad(k, carry):
        j = 4 * k
        for m in range(4):
            dwait(j + m, m)
            scat1(m)
            dload(j + m + NBUF, m)
        return carry

    lax.fori_loop(0, (ANCHUNK - 5) // 4, cquad, 0)
    dwait(120, 0); scat1(0); dload(124, 0)
    dwait(121, 1); scat1(1)
    dwait(122, 2); scat1(2)
    dwait(123, 3); scat1(3)
    dwait(124, 0); scat1(0)
    plsc.subcore_barrier()
    pltpu.sync_copy(acc.at[pl.ds(sid * ZR, ZR)],
                    cnt_hbm.at[cid, pl.ds(sid * ZR, ZR)])

    zcp2 = pltpu.async_copy(zero_hbm.at[pl.ds(sid * ZR, ZR)],
                            acc.at[pl.ds(sid * ZR, ZR)], zsem)

    def iload(j, m):
        e0 = wid * AEPW + j * AC
        pltpu.async_copy(src_hbm.at[pl.ds(e0, AC)], sidx[m], isem[m])
        pltpu.async_copy(dst_hbm.at[pl.ds(e0, AC)], didx[m], isem[m])

    def iwait(j, m):
        e0 = wid * AEPW + j * AC
        pltpu.make_async_copy(src_hbm.at[pl.ds(e0, AC)], sidx[m], isem[m]).wait()
        pltpu.make_async_copy(dst_hbm.at[pl.ds(e0, AC)], didx[m], isem[m]).wait()

    def gissue(m):
        pltpu.async_copy(h_hbm.at[sidx[m]], rows[m], gsem[m])

    def gwait(m):
        pltpu.make_async_copy(h_hbm.at[sidx[m]], rows[m], gsem[m]).wait()

    def scat(m):
        pltpu.sync_copy(rows[m], acc.at[didx[m]], add=True)

    for j in range(NBUF):
        iload(j, j)
    for j in range(NBUF - 1):
        iwait(j, j)
        gissue(j)
    zcp2.wait()
    plsc.subcore_barrier()

    def quad(k, carry):
        j = 4 * k
        for m in range(4):
            gwait(m)
            scat(m)
            iload(j + m + NBUF, m)
            iwait(j + m + NBUF - 1, (m + NBUF - 1) % NBUF)
            gissue((m + NBUF - 1) % NBUF)
        return carry

    lax.fori_loop(0, (ANCHUNK - 5) // 4, quad, 0)
    gwait(0); scat(0)
    iload(124, 0)
    iwait(123, 3)
    gissue(3)
    gwait(1); scat(1)
    iwait(124, 0)
    gissue(0)
    gwait(2); scat(2)
    gwait(3); scat(3)
    gwait(0); scat(0)

    plsc.subcore_barrier()
    pltpu.sync_copy(acc.at[pl.ds(sid * ZR, ZR)],
                    out_hbm.at[cid, pl.ds(sid * ZR, ZR)])


_RB = 5000


def _combine_body(has_bn, h, accp, cntp, w1, w2, b, g, be, out):
    s = accp[0] + accp[1]
    c = cntp[0, :, 0:1] + cntp[1, :, 0:1]
    hn = s * (1.0 / jnp.maximum(c, 1.0))
    h2 = (lax.dot_general(h[...], w1[...], (((1,), (1,)), ((), ())),
                          preferred_element_type=jnp.float32)
          + lax.dot_general(hn, w2[...], (((1,), (1,)), ((), ())),
                            preferred_element_type=jnp.float32)
          + b[...])
    nrm = jnp.maximum(jnp.sqrt(jnp.sum(h2 * h2, axis=1, keepdims=True)), 1e-12)
    y = h2 / nrm
    if has_bn:
        y = y * (g[...] / jnp.sqrt(1.0 + 1e-5)) + be[...]
        y = jnp.maximum(y, 0.0)
    out[...] = y


def _combine(h, acc, cnt, w1, w2, b, g, be, has_bn):
    mat = pl.BlockSpec((D, D), lambda i: (0, 0))
    vec = pl.BlockSpec((1, D), lambda i: (0, 0))
    return pl.pallas_call(
        functools.partial(_combine_body, has_bn),
        grid=(N // _RB,),
        in_specs=[
            pl.BlockSpec((_RB, D), lambda i: (i, 0)),
            pl.BlockSpec((NC, _RB, D), lambda i: (0, i, 0)),
            pl.BlockSpec((NC, _RB, 8), lambda i: (0, i, 0)),
            mat, mat, vec, vec, vec,
        ],
        out_specs=pl.BlockSpec((_RB, D), lambda i: (i, 0)),
        out_shape=jax.ShapeDtypeStruct((N, D), jnp.float32),
    )(h, acc, cnt, w1, w2, b[None, :], g[None, :], be[None, :])


def kernel(x, edge_index, W1_0, W2_0, b_0, W1_1, W2_1, b_1, W1_2, W2_2, b_2,
           g_0, be_0, g_1, be_1):
    ei = edge_index.astype(jnp.int32)
    src = ei[0]
    dst = ei[1]
    zero_nd = jnp.zeros((NP, D), jnp.float32)

    cnt_full, acc = _sc_agg_count(x, src, dst, zero_nd)
    cnt = cnt_full[:, :, :8]
    h = _combine(x, acc, cnt, W1_0, W2_0, b_0, g_0, be_0, True)
    for w1, w2, b, g, be, has_bn in [
        (W1_1, W2_1, b_1, g_1, be_1, True),
        (W1_2, W2_2, b_2, g_1, be_1, False),
    ]:
        acc = _sc_agg(h, src, dst, zero_nd)
        h = _combine(h, acc, cnt, w1, w2, b, g, be, has_bn)
    return h
